# Initial kernel scaffold; baseline (speedup 1.0000x reference)
import jax
import jax.numpy as jnp
from jax.experimental import pallas as pl


def kernel(x, edge_index, W1, b1, W2, b2):
    raise NotImplementedError("write your pallas kernel here")


# speedup vs baseline: 11.1374x; 11.1374x over previous
import functools

import jax
import jax.numpy as jnp
from jax import lax
from jax.experimental import pallas as pl
from jax.experimental.pallas import tpu as pltpu
from jax.experimental.pallas import tpu_sc as plsc

N = 10000
F = 128
HID = 256
E = 320000

NC = 2
NS = 16
NW = NC * NS

NPAD = 10240
STRIPE = NPAD // NS
C = 128
EPW = NPAD
K = EPW // C
EPAD = NW * EPW

ROWB = 1024
GRID = NPAD // ROWB


@functools.cache
def _sc_kernels():
    mesh = plsc.VectorSubcoreMesh(core_axis_name="c", subcore_axis_name="s")

    @functools.partial(
        pl.kernel,
        out_type=jax.ShapeDtypeStruct((NC, NS, STRIPE), jnp.float32),
        mesh=mesh,
        scratch_types=[
            pltpu.VMEM((K, C), jnp.int32),
            pltpu.VMEM((C,), jnp.float32),
            pltpu.VMEM_SHARED((NPAD,), jnp.float32),
        ],
    )
    def sc_degree(dst_hbm, zeros_hbm, ones_hbm, out_hbm, dst_v, ones_v, acc):
        c = lax.axis_index("c")
        s = lax.axis_index("s")
        wid = s * NC + c
        pltpu.sync_copy(dst_hbm.at[wid], dst_v)
        pltpu.sync_copy(ones_hbm, ones_v)
        pltpu.sync_copy(zeros_hbm, acc.at[pl.ds(s * STRIPE, STRIPE)])
        plsc.subcore_barrier()

        @pl.loop(0, K)
        def _(j):
            pltpu.sync_copy(ones_v, acc.at[dst_v.at[j]], add=True)

        plsc.subcore_barrier()
        pltpu.sync_copy(acc.at[pl.ds(s * STRIPE, STRIPE)], out_hbm.at[c, s])

    KP = K // 2

    @functools.partial(
        pl.kernel,
        out_type=jax.ShapeDtypeStruct((NC, NS, STRIPE, F), jnp.float32),
        mesh=mesh,
        scratch_types=[
            pltpu.VMEM((KP, C), jnp.int32),
            pltpu.VMEM((KP, C), jnp.int32),
            pltpu.VMEM((C, F), jnp.float32),
            pltpu.VMEM((C, F), jnp.float32),
            pltpu.VMEM_SHARED((NPAD, F), jnp.float32),
            pltpu.SemaphoreType.DMA,
            pltpu.SemaphoreType.DMA,
        ],
    )
    def sc_aggregate(g_hbm, src_hbm, dst_hbm, zeros_hbm, out_hbm,
                     src_v, dst_v, buf0, buf1, acc, sem0, sem1):
        c = lax.axis_index("c")
        s = lax.axis_index("s")
        wid = s * NC + c
        pltpu.sync_copy(zeros_hbm, acc.at[pl.ds(s * STRIPE, STRIPE)])
        plsc.subcore_barrier()

        bufs = (buf0, buf1)
        sems = (sem0, sem1)
        for p in range(2):
            pltpu.sync_copy(src_hbm.at[wid, pl.ds(p * KP, KP)], src_v)
            pltpu.sync_copy(dst_hbm.at[wid, pl.ds(p * KP, KP)], dst_v)
            pltpu.async_copy(g_hbm.at[src_v.at[0]], buf0, sem0)
            pltpu.async_copy(g_hbm.at[src_v.at[1]], buf1, sem1)

            @pl.loop(0, KP // 2)
            def _(i):
                for b in range(2):
                    j = i * 2 + b
                    pltpu.make_async_copy(
                        g_hbm.at[src_v.at[j]], bufs[b], sems[b]).wait()
                    pltpu.sync_copy(bufs[b], acc.at[dst_v.at[j]], add=True)
                    jn = jnp.where(j + 2 < KP, j + 2, j)
                    pltpu.async_copy(g_hbm.at[src_v.at[jn]], bufs[b], sems[b])

            for b in range(2):
                pltpu.make_async_copy(
                    g_hbm.at[src_v.at[0]], bufs[b], sems[b]).wait()

        plsc.subcore_barrier()
        pltpu.sync_copy(acc.at[pl.ds(s * STRIPE, STRIPE)], out_hbm.at[c, s])

    return sc_degree, sc_aggregate


def _sc_degree(*args):
    return _sc_kernels()[0](*args)


def _sc_aggregate(*args):
    return _sc_kernels()[1](*args)


def _tc_prescale_kernel(d0_ref, d1_ref, x_ref, g_ref, dinv_ref, dinv2_ref):
    deg = d0_ref[...] + d1_ref[...] + 1.0
    dinv = lax.rsqrt(deg)
    dinv2 = 1.0 / deg
    dinv_ref[...] = dinv
    dinv2_ref[...] = dinv2
    g_ref[...] = x_ref[...] * dinv


def _tc_mid_kernel(a0_ref, a1_ref, x_ref, dinv_ref, dinv2_ref,
                   w1_ref, b1_ref, w2_ref, t2_ref, g2_ref):
    dinv = dinv_ref[...]
    p = dinv * (a0_ref[...] + a1_ref[...]) + dinv2_ref[...] * x_ref[...]
    h = jnp.maximum(
        jnp.dot(p, w1_ref[...], preferred_element_type=jnp.float32)
        + b1_ref[...], 0.0)
    t2 = jnp.dot(h, w2_ref[...], preferred_element_type=jnp.float32)
    t2_ref[...] = t2
    g2_ref[...] = t2 * dinv


def _tc_final_kernel(a0_ref, a1_ref, t2_ref, dinv_ref, dinv2_ref, b2_ref,
                     out_ref):
    o = (dinv_ref[...] * (a0_ref[...] + a1_ref[...])
         + dinv2_ref[...] * t2_ref[...] + b2_ref[...])
    m = jnp.max(o, axis=1, keepdims=True)
    e = jnp.exp(o - m)
    lse = jnp.log(jnp.sum(e, axis=1, keepdims=True)) + m
    out_ref[...] = o - lse


def _row_spec(width):
    return pl.BlockSpec((ROWB, width), lambda i: (i, 0))


def _full_spec(shape):
    return pl.BlockSpec(shape, lambda i: tuple(0 for _ in shape))


def kernel(x, edge_index, W1, b1, W2, b2):
    src = edge_index[0].astype(jnp.int32)
    dst = edge_index[1].astype(jnp.int32)
    pad = EPAD - E
    src3 = jnp.concatenate([src, jnp.zeros((pad,), jnp.int32)]).reshape(NW, K, C)
    dst3 = jnp.concatenate([dst, jnp.full((pad,), N, jnp.int32)]).reshape(NW, K, C)

    xp = jnp.zeros((NPAD, F), jnp.float32).at[:N].set(x)
    zeros1 = jnp.zeros((STRIPE,), jnp.float32)
    zeros2 = jnp.zeros((STRIPE, F), jnp.float32)
    ones_c = jnp.ones((C,), jnp.float32)

    degp = _sc_degree(dst3, zeros1, ones_c).reshape(NC, NPAD)
    d0 = degp[0].reshape(NPAD, 1)
    d1 = degp[1].reshape(NPAD, 1)

    g1, dinv, dinv2 = pl.pallas_call(
        _tc_prescale_kernel,
        grid=(GRID,),
        in_specs=[_row_spec(1), _row_spec(1), _row_spec(F)],
        out_specs=[_row_spec(F), _row_spec(1), _row_spec(1)],
        out_shape=[
            jax.ShapeDtypeStruct((NPAD, F), jnp.float32),
            jax.ShapeDtypeStruct((NPAD, 1), jnp.float32),
            jax.ShapeDtypeStruct((NPAD, 1), jnp.float32),
        ],
    )(d0, d1, xp)

    agg1 = _sc_aggregate(g1, src3, dst3, zeros2).reshape(NC, NPAD, F)

    t2, g2 = pl.pallas_call(
        _tc_mid_kernel,
        grid=(GRID,),
        in_specs=[
            _row_spec(F), _row_spec(F), _row_spec(F),
            _row_spec(1), _row_spec(1),
            _full_spec((F, HID)), _full_spec((1, HID)), _full_spec((HID, F)),
        ],
        out_specs=[_row_spec(F), _row_spec(F)],
        out_shape=[
            jax.ShapeDtypeStruct((NPAD, F), jnp.float32),
            jax.ShapeDtypeStruct((NPAD, F), jnp.float32),
        ],
    )(agg1[0], agg1[1], xp, dinv, dinv2, W1, b1.reshape(1, HID), W2)

    agg2 = _sc_aggregate(g2, src3, dst3, zeros2).reshape(NC, NPAD, F)

    out = pl.pallas_call(
        _tc_final_kernel,
        grid=(GRID,),
        in_specs=[
            _row_spec(F), _row_spec(F), _row_spec(F),
            _row_spec(1), _row_spec(1), _full_spec((1, F)),
        ],
        out_specs=_row_spec(F),
        out_shape=jax.ShapeDtypeStruct((NPAD, F), jnp.float32),
    )(agg2[0], agg2[1], t2, dinv, dinv2, b2.reshape(1, F))

    return out[:N]


# speedup vs baseline: 11.1978x; 1.0054x over previous
import functools

import jax
import jax.numpy as jnp
from jax import lax
from jax.experimental import pallas as pl
from jax.experimental.pallas import tpu as pltpu
from jax.experimental.pallas import tpu_sc as plsc

N = 10000
F = 128
HID = 256
E = 320000

NC = 2
NS = 16
NW = NC * NS

NPAD = 10240
STRIPE = NPAD // NS
C = 128
EPW = NPAD
EPAD = NW * EPW
KD = EPW // C
QP = KD // 2
U = 4
CU = C // U

ROWB = 1024
GRID = NPAD // ROWB


@functools.cache
def _sc_kernels():
    mesh = plsc.VectorSubcoreMesh(core_axis_name="c", subcore_axis_name="s")

    @functools.partial(
        pl.kernel,
        out_type=jax.ShapeDtypeStruct((NC, NS, STRIPE), jnp.float32),
        mesh=mesh,
        scratch_types=[
            pltpu.VMEM((KD, 128), jnp.int32),
            pltpu.VMEM((128,), jnp.float32),
            pltpu.VMEM_SHARED((NPAD,), jnp.float32),
        ],
    )
    def sc_degree(dst_hbm, zeros_hbm, ones_hbm, out_hbm, dst_v, ones_v, acc):
        c = lax.axis_index("c")
        s = lax.axis_index("s")
        wid = s * NC + c
        pltpu.sync_copy(dst_hbm.at[wid], dst_v)
        pltpu.sync_copy(ones_hbm, ones_v)
        pltpu.sync_copy(zeros_hbm, acc.at[pl.ds(s * STRIPE, STRIPE)])
        plsc.subcore_barrier()

        @pl.loop(0, KD)
        def _(j):
            pltpu.sync_copy(ones_v, acc.at[dst_v.at[j]], add=True)

        plsc.subcore_barrier()
        pltpu.sync_copy(acc.at[pl.ds(s * STRIPE, STRIPE)], out_hbm.at[c, s])

    @functools.partial(
        pl.kernel,
        out_type=jax.ShapeDtypeStruct((NC, NS, STRIPE, F), jnp.float32),
        mesh=mesh,
        scratch_types=[
            pltpu.VMEM((QP, C), jnp.int32),
            pltpu.VMEM((QP, C), jnp.int32),
            pltpu.VMEM((C, F), jnp.float32),
            pltpu.VMEM((C, F), jnp.float32),
            pltpu.VMEM_SHARED((NPAD, F), jnp.float32),
            pltpu.SemaphoreType.DMA,
            pltpu.SemaphoreType.DMA,
            pltpu.SemaphoreType.DMA,
            pltpu.SemaphoreType.DMA,
        ],
    )
    def sc_aggregate(g_hbm, src_hbm, dst_hbm, zeros_hbm, out_hbm,
                     src_v, dst_v, buf0, buf1, acc, sg0, sg1, ss0, ss1):
        c = lax.axis_index("c")
        s = lax.axis_index("s")
        wid = s * NC + c
        pltpu.sync_copy(zeros_hbm, acc.at[pl.ds(s * STRIPE, STRIPE)])
        plsc.subcore_barrier()

        bufs = (buf0, buf1)
        sem_g = (sg0, sg1)
        sem_s = (ss0, ss1)

        def gathers(q, b):
            for u in range(U):
                pltpu.async_copy(
                    g_hbm.at[src_v.at[q, pl.ds(u * CU, CU)]],
                    bufs[b].at[pl.ds(u * CU, CU)], sem_g[b])

        def wait_gathers(b):
            for u in range(U):
                pltpu.make_async_copy(
                    g_hbm.at[src_v.at[0, pl.ds(0, CU)]],
                    bufs[b].at[pl.ds(0, CU)], sem_g[b]).wait()

        def scatter(q, b):
            pltpu.async_copy(bufs[b], acc.at[dst_v.at[q]], sem_s[b], add=True)

        def wait_scatter(b):
            pltpu.make_async_copy(bufs[b], acc.at[dst_v.at[0]],
                                  sem_s[b]).wait()

        for p in range(2):
            pltpu.sync_copy(src_hbm.at[wid, pl.ds(p * QP, QP)], src_v)
            pltpu.sync_copy(dst_hbm.at[wid, pl.ds(p * QP, QP)], dst_v)
            gathers(0, 0)
            wait_gathers(0)
            scatter(0, 0)
            gathers(1, 1)
            @pl.loop(0, (QP - 2) // 2)
            def _(i):
                for b in (1, 0):
                    q = 2 * i + 2 - b
                    wait_gathers(b)
                    scatter(q, b)
                    wait_scatter(1 - b)
                    gathers(q + 1, 1 - b)
            wait_gathers(1)
            scatter(QP - 1, 1)
            wait_scatter(0)
            wait_scatter(1)

        plsc.subcore_barrier()
        pltpu.sync_copy(acc.at[pl.ds(s * STRIPE, STRIPE)], out_hbm.at[c, s])

    return sc_degree, sc_aggregate


def _sc_degree(*args):
    return _sc_kernels()[0](*args)


def _sc_aggregate(*args):
    return _sc_kernels()[1](*args)


def _tc_prescale_kernel(d0_ref, d1_ref, x_ref, g_ref, dinv_ref, dinv2_ref):
    deg = d0_ref[...] + d1_ref[...] + 1.0
    dinv = lax.rsqrt(deg)
    dinv2 = 1.0 / deg
    dinv_ref[...] = dinv
    dinv2_ref[...] = dinv2
    g_ref[...] = x_ref[...] * dinv


def _tc_mid_kernel(a0_ref, a1_ref, x_ref, dinv_ref, dinv2_ref,
                   w1_ref, b1_ref, w2_ref, t2_ref, g2_ref):
    dinv = dinv_ref[...]
    p = dinv * (a0_ref[...] + a1_ref[...]) + dinv2_ref[...] * x_ref[...]
    h = jnp.maximum(
        jnp.dot(p, w1_ref[...], preferred_element_type=jnp.float32)
        + b1_ref[...], 0.0)
    t2 = jnp.dot(h, w2_ref[...], preferred_element_type=jnp.float32)
    t2_ref[...] = t2
    g2_ref[...] = t2 * dinv


def _tc_final_kernel(a0_ref, a1_ref, t2_ref, dinv_ref, dinv2_ref, b2_ref,
                     out_ref):
    o = (dinv_ref[...] * (a0_ref[...] + a1_ref[...])
         + dinv2_ref[...] * t2_ref[...] + b2_ref[...])
    m = jnp.max(o, axis=1, keepdims=True)
    e = jnp.exp(o - m)
    lse = jnp.log(jnp.sum(e, axis=1, keepdims=True)) + m
    out_ref[...] = o - lse


def _row_spec(width):
    return pl.BlockSpec((ROWB, width), lambda i: (i, 0))


def _full_spec(shape):
    return pl.BlockSpec(shape, lambda i: tuple(0 for _ in shape))


def kernel(x, edge_index, W1, b1, W2, b2):
    src = edge_index[0].astype(jnp.int32)
    dst = edge_index[1].astype(jnp.int32)
    pad = EPAD - E
    srcp = jnp.concatenate([src, jnp.zeros((pad,), jnp.int32)])
    dstp = jnp.concatenate([dst, jnp.full((pad,), N, jnp.int32)])
    src3 = srcp.reshape(NW, KD, C)
    dst3 = dstp.reshape(NW, KD, C)

    xp = jnp.zeros((NPAD, F), jnp.float32).at[:N].set(x)
    zeros1 = jnp.zeros((STRIPE,), jnp.float32)
    zeros2 = jnp.zeros((STRIPE, F), jnp.float32)
    ones_c = jnp.ones((128,), jnp.float32)

    degp = _sc_degree(dst3, zeros1, ones_c).reshape(NC, NPAD)
    d0 = degp[0].reshape(NPAD, 1)
    d1 = degp[1].reshape(NPAD, 1)

    g1, dinv, dinv2 = pl.pallas_call(
        _tc_prescale_kernel,
        grid=(GRID,),
        in_specs=[_row_spec(1), _row_spec(1), _row_spec(F)],
        out_specs=[_row_spec(F), _row_spec(1), _row_spec(1)],
        out_shape=[
            jax.ShapeDtypeStruct((NPAD, F), jnp.float32),
            jax.ShapeDtypeStruct((NPAD, 1), jnp.float32),
            jax.ShapeDtypeStruct((NPAD, 1), jnp.float32),
        ],
    )(d0, d1, xp)

    agg1 = _sc_aggregate(g1, src3, dst3, zeros2).reshape(NC, NPAD, F)

    t2, g2 = pl.pallas_call(
        _tc_mid_kernel,
        grid=(GRID,),
        in_specs=[
            _row_spec(F), _row_spec(F), _row_spec(F),
            _row_spec(1), _row_spec(1),
            _full_spec((F, HID)), _full_spec((1, HID)), _full_spec((HID, F)),
        ],
        out_specs=[_row_spec(F), _row_spec(F)],
        out_shape=[
            jax.ShapeDtypeStruct((NPAD, F), jnp.float32),
            jax.ShapeDtypeStruct((NPAD, F), jnp.float32),
        ],
    )(agg1[0], agg1[1], xp, dinv, dinv2, W1, b1.reshape(1, HID), W2)

    agg2 = _sc_aggregate(g2, src3, dst3, zeros2).reshape(NC, NPAD, F)

    out = pl.pallas_call(
        _tc_final_kernel,
        grid=(GRID,),
        in_specs=[
            _row_spec(F), _row_spec(F), _row_spec(F),
            _row_spec(1), _row_spec(1), _full_spec((1, F)),
        ],
        out_specs=_row_spec(F),
        out_shape=jax.ShapeDtypeStruct((NPAD, F), jnp.float32),
    )(agg2[0], agg2[1], t2, dinv, dinv2, b2.reshape(1, F))

    return out[:N]
